# initial kernel scaffold (unmeasured)
import jax
import jax.numpy as jnp
from jax import lax
from jax.experimental import pallas as pl
from jax.experimental.pallas import tpu as pltpu


def kernel(
    x,
):
    def body(*refs):
        pass

    out_shape = jax.ShapeDtypeStruct(..., jnp.float32)
    return pl.pallas_call(body, out_shape=out_shape)(...)


# baseline (device time: 11436 ns/iter reference)
import jax
import jax.numpy as jnp
from jax import lax
from jax.experimental import pallas as pl
from jax.experimental.pallas import tpu as pltpu

N_DEV = 4


def kernel(x):
    m_per, n = x.shape

    def body(x_ref, out_ref, comm_ref, send_sems, recv_sems):
        my_pos = lax.axis_index("i")
        left = (my_pos - 1) % N_DEV
        right = (my_pos + 1) % N_DEV

        barrier_sem = pltpu.get_barrier_semaphore()
        for nbr in [left, right]:
            pl.semaphore_signal(
                barrier_sem, inc=1,
                device_id=(nbr,), device_id_type=pl.DeviceIdType.MESH,
            )
        pl.semaphore_wait(barrier_sem, 2)

        xv = x_ref[:, :]

        y = xv
        d = 1
        while d < m_per:
            shifted = jnp.concatenate(
                [jnp.ones((d, n), jnp.float32), y[: m_per - d]], axis=0
            )
            y = y * shifted
            d *= 2

        comm_ref[0, :, :] = y[m_per - 1 :, :]

        for h in range(N_DEV - 1):
            rdma = pltpu.make_async_remote_copy(
                src_ref=comm_ref.at[h],
                dst_ref=comm_ref.at[h + 1],
                send_sem=send_sems.at[h],
                recv_sem=recv_sems.at[h + 1],
                device_id=(right,),
                device_id_type=pl.DeviceIdType.MESH,
            )
            rdma.start()
            rdma.wait()

        prefix = jnp.ones((1, n), jnp.float32)
        for s in range(1, N_DEV):
            tot_s = comm_ref[s, :, :]
            prefix = prefix * jnp.where(s <= my_pos, tot_s, 1.0)

        out_ref[:, :] = y * prefix

    return pl.pallas_call(
        body,
        out_shape=jax.ShapeDtypeStruct((m_per, n), x.dtype),
        in_specs=[pl.BlockSpec(memory_space=pltpu.VMEM)],
        out_specs=pl.BlockSpec(memory_space=pltpu.VMEM),
        scratch_shapes=[
            pltpu.VMEM((N_DEV, 1, n), jnp.float32),
            pltpu.SemaphoreType.DMA((N_DEV - 1,)),
            pltpu.SemaphoreType.DMA((N_DEV,)),
        ],
        compiler_params=pltpu.CompilerParams(collective_id=0),
    )(x)


# device time: 9199 ns/iter; 1.2432x vs baseline; 1.2432x over previous
import functools

import jax
import jax.numpy as jnp
from jax import lax
from jax.experimental import pallas as pl
from jax.experimental.pallas import tpu as pltpu

N_DEV = 4


def kernel(x):
    m_per, n = x.shape

    def body(x_ref, out_ref, tot_ref, comm_ref, send_sems, recv_sems):
        my_pos = lax.axis_index("i")
        left = (my_pos - 1) % N_DEV
        right = (my_pos + 1) % N_DEV

        barrier_sem = pltpu.get_barrier_semaphore()
        for nbr in [left, right]:
            pl.semaphore_signal(
                barrier_sem, inc=1,
                device_id=(nbr,), device_id_type=pl.DeviceIdType.MESH,
            )
        pl.semaphore_wait(barrier_sem, 2)

        xv = x_ref[:, :]

        t = xv
        r = m_per
        while r > 1:
            t = t[: r // 2] * t[r // 2 :]
            r //= 2
        tot_ref[:, :] = t

        def desc(src, dst):
            return pltpu.make_async_remote_copy(
                src_ref=tot_ref,
                dst_ref=comm_ref.at[src],
                send_sem=send_sems.at[dst],
                recv_sem=recv_sems.at[src],
                device_id=(dst,),
                device_id_type=pl.DeviceIdType.MESH,
            )

        pairs = [(s, d) for s in range(N_DEV) for d in range(s + 1, N_DEV)]

        for src, dst in pairs:
            @pl.when(my_pos == src)
            def _(src=src, dst=dst):
                desc(src, dst).start()

        y = xv
        d = 1
        while d < m_per:
            shifted = jnp.concatenate(
                [jnp.ones((d, n), jnp.float32), y[: m_per - d]], axis=0
            )
            y = y * shifted
            d *= 2

        for src, dst in pairs:
            @pl.when(my_pos == dst)
            def _(src=src, dst=dst):
                desc(src, dst).wait_recv()

            @pl.when(my_pos == src)
            def _(src=src, dst=dst):
                desc(src, dst).wait_send()

        prefix = jnp.ones((1, n), jnp.float32)
        for s in range(N_DEV - 1):
            prefix = prefix * jnp.where(s < my_pos, comm_ref[s, :, :], 1.0)

        out_ref[:, :] = y * prefix

        @functools.partial(
            pl.run_scoped, second_barrier=pltpu.SemaphoreType.REGULAR
        )
        def _(second_barrier):
            for nbr in [left, right]:
                pl.semaphore_signal(
                    second_barrier, inc=1,
                    device_id=(nbr,), device_id_type=pl.DeviceIdType.MESH,
                )
            pl.semaphore_wait(second_barrier, 2)

    return pl.pallas_call(
        body,
        out_shape=jax.ShapeDtypeStruct((m_per, n), x.dtype),
        in_specs=[pl.BlockSpec(memory_space=pltpu.VMEM)],
        out_specs=pl.BlockSpec(memory_space=pltpu.VMEM),
        scratch_shapes=[
            pltpu.VMEM((1, n), jnp.float32),
            pltpu.VMEM((N_DEV, 1, n), jnp.float32),
            pltpu.SemaphoreType.DMA((N_DEV,)),
            pltpu.SemaphoreType.DMA((N_DEV,)),
        ],
        compiler_params=pltpu.CompilerParams(collective_id=0),
    )(x)
